# baseline (speedup 1.0000x reference)
import functools

import jax
import jax.numpy as jnp
from jax.experimental import pallas as pl

_LANES = 128


def _fused_body(xyz_t_ref, xyz_s_ref, sq_t_ref, sq_s_ref,
                feat_t_ref, feat_s_ref,
                w1a_ref, w1b_ref, b1_ref, w2_ref, b2_ref, out_ref):
    xyz_t = xyz_t_ref[0]
    xyz_s = xyz_s_ref[0]
    bn = xyz_t.shape[0]
    m = xyz_s.shape[0]

    cross = jax.lax.dot_general(
        xyz_t, xyz_s, (((1,), (1,)), ((), ())),
        preferred_element_type=jnp.float32)
    d2 = jnp.maximum(sq_t_ref[0] + sq_s_ref[0] - 2.0 * cross, 0.0)

    iota_m = jax.lax.broadcasted_iota(jnp.int32, (bn, m), 1)
    key = jax.lax.bitcast_convert_type(
        (jax.lax.bitcast_convert_type(d2 + 1e-30, jnp.int32) & ~(m - 1))
        | iota_m,
        jnp.float32)
    big = jnp.float32(3.4e38)

    nchunk = m // _LANES
    b1 = key[:, 0:_LANES]
    b2 = jnp.full((bn, _LANES), big)
    b3 = jnp.full((bn, _LANES), big)
    for c in range(1, nchunk):
        x = key[:, c * _LANES:(c + 1) * _LANES]
        hi1 = jnp.maximum(b1, x)
        b1 = jnp.minimum(b1, x)
        hi2 = jnp.maximum(b2, hi1)
        b2 = jnp.minimum(b2, hi1)
        b3 = jnp.minimum(b3, hi2)

    m1 = jnp.min(b1, axis=1, keepdims=True)
    b1x = jnp.where(b1 == m1, big, b1)
    m2 = jnp.minimum(jnp.min(b1x, axis=1, keepdims=True),
                     jnp.min(b2, axis=1, keepdims=True))
    m3 = jnp.min(jnp.minimum(jnp.where(b1x == m2, big, b1x),
                             jnp.minimum(jnp.where(b2 == m2, big, b2), b3)),
                 axis=1, keepdims=True)

    def unpack_rk(k):
        dk = jax.lax.bitcast_convert_type(
            jax.lax.bitcast_convert_type(k, jnp.int32) & ~(m - 1), jnp.float32)
        return 1.0 / (dk + 1e-8)

    total = unpack_rk(m1) + unpack_rk(m2) + unpack_rk(m3)
    inv_total = 1.0 / total
    sparse_w = jnp.where(key <= m3, unpack_rk(key) * inv_total, 0.0)

    interp = jax.lax.dot_general(
        sparse_w, feat_s_ref[0], (((1,), (0,)), ((), ())),
        preferred_element_type=jnp.float32)

    h = jax.nn.relu(
        jax.lax.dot_general(interp, w1a_ref[...], (((1,), (0,)), ((), ())),
                            preferred_element_type=jnp.float32)
        + jax.lax.dot_general(feat_t_ref[0], w1b_ref[...], (((1,), (0,)), ((), ())),
                              preferred_element_type=jnp.float32)
        + b1_ref[...])
    out_ref[0] = (
        jax.lax.dot_general(h, w2_ref[...], (((1,), (0,)), ((), ())),
                            preferred_element_type=jnp.float32)
        + b2_ref[...])


@functools.partial(jax.jit, static_argnames=("bn",))
def _fused(xyz_target, xyz_src, sq_t, sq_s, feat_target, feat_src,
           W1a, W1b, b1, W2, b2, bn=4096):
    B, N, _ = xyz_target.shape
    M = xyz_src.shape[1]
    C1 = feat_target.shape[2]
    C2 = feat_src.shape[2]
    Cout = W2.shape[1]
    grid = (B, N // bn)
    return pl.pallas_call(
        _fused_body,
        grid=grid,
        in_specs=[
            pl.BlockSpec((1, bn, 3), lambda b, i: (b, i, 0)),
            pl.BlockSpec((1, M, 3), lambda b, i: (b, 0, 0)),
            pl.BlockSpec((1, bn, 1), lambda b, i: (b, i, 0)),
            pl.BlockSpec((1, 1, M), lambda b, i: (b, 0, 0)),
            pl.BlockSpec((1, bn, C1), lambda b, i: (b, i, 0)),
            pl.BlockSpec((1, M, C2), lambda b, i: (b, 0, 0)),
            pl.BlockSpec((C2, Cout), lambda b, i: (0, 0)),
            pl.BlockSpec((C1, Cout), lambda b, i: (0, 0)),
            pl.BlockSpec((1, Cout), lambda b, i: (0, 0)),
            pl.BlockSpec((Cout, Cout), lambda b, i: (0, 0)),
            pl.BlockSpec((1, Cout), lambda b, i: (0, 0)),
        ],
        out_specs=pl.BlockSpec((1, bn, Cout), lambda b, i: (b, i, 0)),
        out_shape=jax.ShapeDtypeStruct((B, N, Cout), jnp.float32),
    )(xyz_target, xyz_src, sq_t, sq_s, feat_target, feat_src,
      W1a, W1b, b1, W2, b2)


def kernel(xyz_target, xyz_src, feat_target, feat_src, W1, b1, W2, b2):
    C2 = feat_src.shape[2]
    W1a = W1[:C2]
    W1b = W1[C2:]
    sq_t = jnp.sum(xyz_target * xyz_target, axis=-1, keepdims=True)
    sq_s = jnp.sum(xyz_src * xyz_src, axis=-1)[:, None, :]
    return _fused(xyz_target, xyz_src, sq_t, sq_s, feat_target, feat_src,
                  W1a, W1b, b1.reshape(1, -1), W2, b2.reshape(1, -1))


# speedup vs baseline: 1.0614x; 1.0614x over previous
import functools

import jax
import jax.numpy as jnp
from jax.experimental import pallas as pl

_LANES = 128


def _fused_body(xyz_t_ref, xyz_s_ref, sq_t_ref, sq_s_ref,
                feat_t_ref, feat_s_ref,
                w1a_ref, w1b_ref, b1_ref, w2_ref, b2_ref, out_ref):
    xyz_t = xyz_t_ref[0]
    xyz_s = xyz_s_ref[0]
    bn = xyz_t.shape[0]
    m = xyz_s.shape[0]

    cross = jax.lax.dot_general(
        xyz_t, xyz_s, (((1,), (1,)), ((), ())),
        preferred_element_type=jnp.float32)
    d2 = jnp.maximum(sq_t_ref[0] + sq_s_ref[0] - 2.0 * cross, 1e-30)

    iota_m = jax.lax.broadcasted_iota(jnp.int32, (bn, m), 1)
    key = jax.lax.bitcast_convert_type(
        (jax.lax.bitcast_convert_type(d2, jnp.int32) & ~(m - 1)) | iota_m,
        jnp.float32)
    big = jnp.float32(3.4e38)

    nchunk = m // _LANES
    chunks = [key[:, c * _LANES:(c + 1) * _LANES] for c in range(nchunk)]
    s2 = [(jnp.minimum(a, c), jnp.maximum(a, c))
          for a, c in zip(chunks[0::2], chunks[1::2])]
    s3 = []
    for (a1, a2), (c1, c2) in zip(s2[0::2], s2[1::2]):
        t1 = jnp.maximum(a1, c1)
        t2 = jnp.minimum(a2, c2)
        s3.append((jnp.minimum(a1, c1), jnp.minimum(t1, t2),
                   jnp.maximum(t1, t2)))
    while len(s3) > 1:
        (a1, a2, a3), (c1, c2, c3) = s3[0], s3[1]
        u = jnp.maximum(a1, c1)
        v = jnp.minimum(a2, c2)
        merged = (jnp.minimum(a1, c1), jnp.minimum(u, v),
                  jnp.minimum(jnp.maximum(u, v), jnp.minimum(a3, c3)))
        s3 = s3[2:] + [merged]
    b1, b2, b3 = s3[0]

    m1 = jnp.min(b1, axis=1, keepdims=True)
    b1x = jnp.where(b1 == m1, big, b1)
    m2 = jnp.minimum(jnp.min(b1x, axis=1, keepdims=True),
                     jnp.min(b2, axis=1, keepdims=True))
    m3 = jnp.min(jnp.minimum(jnp.where(b1x == m2, big, b1x),
                             jnp.minimum(jnp.where(b2 == m2, big, b2), b3)),
                 axis=1, keepdims=True)

    def unpack_rk(k):
        return 1.0 / (k + 1e-8)

    total = unpack_rk(m1) + unpack_rk(m2) + unpack_rk(m3)
    inv_total = 1.0 / total
    sparse_w = jnp.where(key <= m3, unpack_rk(key) * inv_total, 0.0)

    interp = jax.lax.dot_general(
        sparse_w, feat_s_ref[0], (((1,), (0,)), ((), ())),
        preferred_element_type=jnp.float32)

    h = jax.nn.relu(
        jax.lax.dot_general(interp, w1a_ref[...], (((1,), (0,)), ((), ())),
                            preferred_element_type=jnp.float32)
        + jax.lax.dot_general(feat_t_ref[0], w1b_ref[...], (((1,), (0,)), ((), ())),
                              preferred_element_type=jnp.float32)
        + b1_ref[...])
    out_ref[0] = (
        jax.lax.dot_general(h, w2_ref[...], (((1,), (0,)), ((), ())),
                            preferred_element_type=jnp.float32)
        + b2_ref[...])


@functools.partial(jax.jit, static_argnames=("bn",))
def _fused(xyz_target, xyz_src, sq_t, sq_s, feat_target, feat_src,
           W1a, W1b, b1, W2, b2, bn=2048):
    B, N, _ = xyz_target.shape
    M = xyz_src.shape[1]
    C1 = feat_target.shape[2]
    C2 = feat_src.shape[2]
    Cout = W2.shape[1]
    grid = (B, N // bn)
    return pl.pallas_call(
        _fused_body,
        grid=grid,
        in_specs=[
            pl.BlockSpec((1, bn, 3), lambda b, i: (b, i, 0)),
            pl.BlockSpec((1, M, 3), lambda b, i: (b, 0, 0)),
            pl.BlockSpec((1, bn, 1), lambda b, i: (b, i, 0)),
            pl.BlockSpec((1, 1, M), lambda b, i: (b, 0, 0)),
            pl.BlockSpec((1, bn, C1), lambda b, i: (b, i, 0)),
            pl.BlockSpec((1, M, C2), lambda b, i: (b, 0, 0)),
            pl.BlockSpec((C2, Cout), lambda b, i: (0, 0)),
            pl.BlockSpec((C1, Cout), lambda b, i: (0, 0)),
            pl.BlockSpec((1, Cout), lambda b, i: (0, 0)),
            pl.BlockSpec((Cout, Cout), lambda b, i: (0, 0)),
            pl.BlockSpec((1, Cout), lambda b, i: (0, 0)),
        ],
        out_specs=pl.BlockSpec((1, bn, Cout), lambda b, i: (b, i, 0)),
        out_shape=jax.ShapeDtypeStruct((B, N, Cout), jnp.float32),
    )(xyz_target, xyz_src, sq_t, sq_s, feat_target, feat_src,
      W1a, W1b, b1, W2, b2)


def kernel(xyz_target, xyz_src, feat_target, feat_src, W1, b1, W2, b2):
    C2 = feat_src.shape[2]
    W1a = W1[:C2]
    W1b = W1[C2:]
    sq_t = jnp.sum(xyz_target * xyz_target, axis=-1, keepdims=True)
    sq_s = jnp.sum(xyz_src * xyz_src, axis=-1)[:, None, :]
    return _fused(xyz_target, xyz_src, sq_t, sq_s, feat_target, feat_src,
                  W1a, W1b, b1.reshape(1, -1), W2, b2.reshape(1, -1))


# speedup vs baseline: 1.0713x; 1.0094x over previous
import functools

import jax
import jax.numpy as jnp
from jax.experimental import pallas as pl

_LANES = 128


def _fused_body(xyz_t_ref, xyz_s_ref, sq_t_ref, sq_s_ref,
                feat_t_ref, feat_s_ref,
                w1a_ref, w1b_ref, b1_ref, w2_ref, b2_ref, out_ref):
    xyz_t = xyz_t_ref[0]
    xyz_s = xyz_s_ref[0]
    bn = xyz_t.shape[0]
    m = xyz_s.shape[0]

    cross = jax.lax.dot_general(
        xyz_t, xyz_s, (((1,), (1,)), ((), ())),
        preferred_element_type=jnp.float32)
    d2 = jnp.maximum(sq_t_ref[0] + sq_s_ref[0] - 2.0 * cross, 1e-30)

    iota_m = jax.lax.broadcasted_iota(jnp.int32, (bn, m), 1)
    key = jax.lax.bitcast_convert_type(
        (jax.lax.bitcast_convert_type(d2, jnp.int32) & ~(m - 1)) | iota_m,
        jnp.float32)
    big = jnp.float32(3.4e38)

    nchunk = m // _LANES
    chunks = [key[:, c * _LANES:(c + 1) * _LANES] for c in range(nchunk)]
    s2 = [(jnp.minimum(a, c), jnp.maximum(a, c))
          for a, c in zip(chunks[0::2], chunks[1::2])]
    s3 = []
    for (a1, a2), (c1, c2) in zip(s2[0::2], s2[1::2]):
        t1 = jnp.maximum(a1, c1)
        t2 = jnp.minimum(a2, c2)
        s3.append((jnp.minimum(a1, c1), jnp.minimum(t1, t2),
                   jnp.maximum(t1, t2)))
    while len(s3) > 1:
        (a1, a2, a3), (c1, c2, c3) = s3[0], s3[1]
        u = jnp.maximum(a1, c1)
        v = jnp.minimum(a2, c2)
        merged = (jnp.minimum(a1, c1), jnp.minimum(u, v),
                  jnp.minimum(jnp.maximum(u, v), jnp.minimum(a3, c3)))
        s3 = s3[2:] + [merged]
    b1, b2, b3 = s3[0]

    m1 = jnp.min(b1, axis=1, keepdims=True)
    b1x = jnp.where(b1 == m1, big, b1)
    m2 = jnp.minimum(jnp.min(b1x, axis=1, keepdims=True),
                     jnp.min(b2, axis=1, keepdims=True))
    m3 = jnp.min(jnp.minimum(jnp.where(b1x == m2, big, b1x),
                             jnp.minimum(jnp.where(b2 == m2, big, b2), b3)),
                 axis=1, keepdims=True)

    def unpack_rk(k):
        return 1.0 / (k + 1e-8)

    total = unpack_rk(m1) + unpack_rk(m2) + unpack_rk(m3)
    inv_total = 1.0 / total
    sparse_w = jnp.where(key <= m3, unpack_rk(key), 0.0)

    interp = inv_total * jax.lax.dot_general(
        sparse_w, feat_s_ref[0], (((1,), (0,)), ((), ())),
        preferred_element_type=jnp.float32)

    h = jax.nn.relu(
        jax.lax.dot_general(interp, w1a_ref[...], (((1,), (0,)), ((), ())),
                            preferred_element_type=jnp.float32)
        + jax.lax.dot_general(feat_t_ref[0], w1b_ref[...], (((1,), (0,)), ((), ())),
                              preferred_element_type=jnp.float32)
        + b1_ref[...])
    out_ref[0] = (
        jax.lax.dot_general(h, w2_ref[...], (((1,), (0,)), ((), ())),
                            preferred_element_type=jnp.float32)
        + b2_ref[...])


@functools.partial(jax.jit, static_argnames=("bn",))
def _fused(xyz_target, xyz_src, sq_t, sq_s, feat_target, feat_src,
           W1a, W1b, b1, W2, b2, bn=2048):
    B, N, _ = xyz_target.shape
    M = xyz_src.shape[1]
    C1 = feat_target.shape[2]
    C2 = feat_src.shape[2]
    Cout = W2.shape[1]
    grid = (B, N // bn)
    return pl.pallas_call(
        _fused_body,
        grid=grid,
        in_specs=[
            pl.BlockSpec((1, bn, 3), lambda b, i: (b, i, 0)),
            pl.BlockSpec((1, M, 3), lambda b, i: (b, 0, 0)),
            pl.BlockSpec((1, bn, 1), lambda b, i: (b, i, 0)),
            pl.BlockSpec((1, 1, M), lambda b, i: (b, 0, 0)),
            pl.BlockSpec((1, bn, C1), lambda b, i: (b, i, 0)),
            pl.BlockSpec((1, M, C2), lambda b, i: (b, 0, 0)),
            pl.BlockSpec((C2, Cout), lambda b, i: (0, 0)),
            pl.BlockSpec((C1, Cout), lambda b, i: (0, 0)),
            pl.BlockSpec((1, Cout), lambda b, i: (0, 0)),
            pl.BlockSpec((Cout, Cout), lambda b, i: (0, 0)),
            pl.BlockSpec((1, Cout), lambda b, i: (0, 0)),
        ],
        out_specs=pl.BlockSpec((1, bn, Cout), lambda b, i: (b, i, 0)),
        out_shape=jax.ShapeDtypeStruct((B, N, Cout), jnp.float32),
    )(xyz_target, xyz_src, sq_t, sq_s, feat_target, feat_src,
      W1a, W1b, b1, W2, b2)


def kernel(xyz_target, xyz_src, feat_target, feat_src, W1, b1, W2, b2):
    C2 = feat_src.shape[2]
    W1a = W1[:C2]
    W1b = W1[C2:]
    sq_t = jnp.sum(xyz_target * xyz_target, axis=-1, keepdims=True)
    sq_s = jnp.sum(xyz_src * xyz_src, axis=-1)[:, None, :]
    return _fused(xyz_target, xyz_src, sq_t, sq_s, feat_target, feat_src,
                  W1a, W1b, b1.reshape(1, -1), W2, b2.reshape(1, -1))
